# baseline (speedup 1.0000x reference)
import functools

import jax
import jax.numpy as jnp
from jax import lax
from jax.experimental import pallas as pl
from jax.experimental.pallas import tpu as pltpu
from jax.experimental.pallas import tpu_sc as plsc

_NC, _NS = 2, 16
_NW = _NC * _NS
_C = 128
_NB = 5


@functools.partial(jax.jit, static_argnums=(2, 3))
def _sc_gather(table, flat_idx, n, d):
    n_per_w = n // _NW
    nch = n_per_w // _C
    assert n_per_w % _C == 0 and nch % _NB == 0

    mesh = plsc.VectorSubcoreMesh(core_axis_name="c", subcore_axis_name="s")

    @functools.partial(
        pl.kernel,
        out_type=jax.ShapeDtypeStruct((n, d), table.dtype),
        mesh=mesh,
        scratch_types=[
            pltpu.VMEM((_NB, _C), jnp.int32),
            pltpu.VMEM((_NB, _C, d), table.dtype),
            pltpu.SemaphoreType.DMA((_NB,)),
            pltpu.SemaphoreType.DMA((_NB,)),
        ],
    )
    def gather_kernel(table_hbm, idx_hbm, out_hbm, idx_v, rows_v, gsem, osem):
        wid = lax.axis_index("s") * _NC + lax.axis_index("c")
        base = wid * n_per_w

        @pl.loop(0, nch, step=_NB)
        def _(k):
            for p in range(_NB):
                off = base + (k + p) * _C

                @pl.when(k + p >= _NB)
                def _():
                    pltpu.make_async_copy(
                        rows_v.at[p],
                        out_hbm.at[pl.ds(off - _NB * _C, _C)],
                        osem.at[p],
                    ).wait()

                pltpu.sync_copy(idx_hbm.at[pl.ds(off, _C)], idx_v.at[p])
                pltpu.make_async_copy(
                    table_hbm.at[idx_v.at[p]], rows_v.at[p], gsem.at[p]
                ).start()

            for p in range(_NB):
                off = base + (k + p) * _C
                pltpu.make_async_copy(
                    table_hbm.at[idx_v.at[p]], rows_v.at[p], gsem.at[p]
                ).wait()
                pltpu.make_async_copy(
                    rows_v.at[p], out_hbm.at[pl.ds(off, _C)], osem.at[p]
                ).start()

        for p in range(_NB):
            off = base + (nch - _NB + p) * _C
            pltpu.make_async_copy(
                rows_v.at[p], out_hbm.at[pl.ds(off, _C)], osem.at[p]
            ).wait()

    return gather_kernel(table, flat_idx)


def kernel(input_batch, seq_lengths, targets_batch, table):
    B, L = input_batch.shape
    V, D = table.shape

    lengths = jnp.maximum(seq_lengths, 1)
    perm = jnp.argsort(-lengths)
    sorted_lengths = lengths[perm]

    pos = jnp.arange(L, dtype=jnp.int32)[None, :]
    tokens = jnp.where(
        pos < sorted_lengths[:, None],
        input_batch[perm].astype(jnp.int32),
        0,
    )
    flat_idx = tokens.reshape(B * L)

    embedded = _sc_gather(table, flat_idx, B * L, D).reshape(B, L, D)
    return embedded, sorted_lengths.astype(jnp.float32), targets_batch[perm]


# speedup vs baseline: 28.9712x; 28.9712x over previous
import functools

import jax
import jax.numpy as jnp
from jax import lax
from jax.experimental import pallas as pl
from jax.experimental.pallas import tpu as pltpu
from jax.experimental.pallas import tpu_sc as plsc

_NC, _NS = 2, 16
_NW = _NC * _NS
_C = 128
_NB = 5


@functools.partial(jax.jit, static_argnums=(2, 3))
def _sc_gather(table, flat_idx, n, d):
    n_per_w = n // _NW
    nch = n_per_w // _C
    assert n_per_w % _C == 0 and nch % _NB == 0

    mesh = plsc.VectorSubcoreMesh(core_axis_name="c", subcore_axis_name="s")

    @functools.partial(
        pl.kernel,
        out_type=jax.ShapeDtypeStruct((n, d), table.dtype),
        mesh=mesh,
        scratch_types=[
            pltpu.VMEM((_NB, _C), jnp.int32),
            pltpu.VMEM((_NB, _C, d), table.dtype),
            pltpu.SemaphoreType.DMA((_NB,)),
            pltpu.SemaphoreType.DMA((_NB,)),
        ],
    )
    def gather_kernel(table_hbm, idx_hbm, out_hbm, idx_v, rows_v, gsem, osem):
        wid = lax.axis_index("s") * _NC + lax.axis_index("c")
        base = wid * n_per_w

        @pl.loop(0, nch, step=_NB)
        def _(k):
            for p in range(_NB):
                off = base + (k + p) * _C

                @pl.when(k + p >= _NB)
                def _():
                    pltpu.make_async_copy(
                        rows_v.at[p],
                        out_hbm.at[pl.ds(off - _NB * _C, _C)],
                        osem.at[p],
                    ).wait()

                pltpu.sync_copy(idx_hbm.at[pl.ds(off, _C)], idx_v.at[p])
                pltpu.make_async_copy(
                    table_hbm.at[idx_v.at[p]], rows_v.at[p], gsem.at[p]
                ).start()

            for p in range(_NB):
                off = base + (k + p) * _C
                pltpu.make_async_copy(
                    table_hbm.at[idx_v.at[p]], rows_v.at[p], gsem.at[p]
                ).wait()
                pltpu.make_async_copy(
                    rows_v.at[p], out_hbm.at[pl.ds(off, _C)], osem.at[p]
                ).start()

        for p in range(_NB):
            off = base + (nch - _NB + p) * _C
            pltpu.make_async_copy(
                rows_v.at[p], out_hbm.at[pl.ds(off, _C)], osem.at[p]
            ).wait()

    return gather_kernel(table, flat_idx)


_Z = 4096


def kernel(input_batch, seq_lengths, targets_batch, table):
    B, L = input_batch.shape
    V, D = table.shape

    lengths = jnp.maximum(seq_lengths, 1)
    perm = jnp.argsort(-lengths)
    sorted_lengths = lengths[perm]

    table_aug = jnp.concatenate(
        [table, jnp.zeros((_Z, D), table.dtype)], axis=0
    )

    pos = jnp.arange(L, dtype=jnp.int32)[None, :]
    flat_pos = jnp.arange(B * L, dtype=jnp.int32).reshape(B, L)
    tokens = jnp.where(
        pos < sorted_lengths[:, None],
        input_batch[perm].astype(jnp.int32),
        V + (flat_pos & (_Z - 1)),
    )
    flat_idx = tokens.reshape(B * L)

    embedded = _sc_gather(table_aug, flat_idx, B * L, D).reshape(B, L, D)
    return embedded, sorted_lengths.astype(jnp.float32), targets_batch[perm]
